# Initial kernel scaffold; baseline (speedup 1.0000x reference)
import jax
import jax.numpy as jnp
from jax.experimental import pallas as pl


def kernel(x, emb, W_ih, W_hh, b_ih, b_hh, W_lin, b_lin):
    raise NotImplementedError("write your pallas kernel here")


# speedup vs baseline: 1.9006x; 1.9006x over previous
import functools

import jax
import jax.numpy as jnp
from jax import lax
from jax.experimental import pallas as pl
from jax.experimental.pallas import tpu as pltpu
from jax.experimental.pallas import tpu_sc as plsc

V = 100000
D = 64
H = 100
C = 6
B = 4096
L = 50

NC = 2
NS = 16
NW = NC * NS
R = B * L
ROWS_PER_W = R // NW
CH = 128
NCH = ROWS_PER_W // CH

BB = 512
NB = B // BB
HP = 128
GP = 4 * HP
KP = D + HP


def _sc_gather(idx, emb):
    mesh = plsc.VectorSubcoreMesh(core_axis_name="c", subcore_axis_name="s")

    @functools.partial(
        pl.kernel,
        mesh=mesh,
        out_type=jax.ShapeDtypeStruct((R, D), jnp.float32),
        scratch_types=[
            pltpu.VMEM((NCH, CH), jnp.int32),
            pltpu.VMEM((CH, D), jnp.float32),
            pltpu.SemaphoreType.DMA,
        ],
        compiler_params=pltpu.CompilerParams(use_tc_tiling_on_sc=False),
    )
    def k(idx_hbm, emb_hbm, out_hbm, idx_v, buf, sem):
        wid = lax.axis_index("s") * NC + lax.axis_index("c")
        base = pl.multiple_of(wid * ROWS_PER_W, CH)
        pltpu.sync_copy(idx_hbm.at[wid], idx_v)

        def body(j, carry):
            pltpu.async_copy(emb_hbm.at[idx_v.at[j]], buf, sem).wait()
            pltpu.sync_copy(buf, out_hbm.at[pl.ds(base + j * CH, CH)])
            return carry

        lax.fori_loop(0, NCH, body, 0)

    return k(idx, emb)


def _sigmoid(x):
    return 0.5 * jnp.tanh(0.5 * x) + 0.5


def _lstm_body(xs_ref, Wc_ref, b_ref, Wl_ref, bl_ref, out_ref, h_ref, c_ref):
    t = pl.program_id(1)

    @pl.when(t == 0)
    def _init():
        h_ref[...] = jnp.zeros_like(h_ref)
        c_ref[...] = jnp.zeros_like(c_ref)

    xt = xs_ref[0]
    h = h_ref[...]
    xh = jnp.concatenate([xt, h], axis=1)
    gates = lax.dot_general(
        xh, Wc_ref[...], (((1,), (0,)), ((), ())),
        preferred_element_type=jnp.float32,
        precision=lax.Precision.HIGHEST,
    ) + b_ref[...]
    i = _sigmoid(gates[:, 0:HP])
    f = _sigmoid(gates[:, HP:2 * HP])
    g = jnp.tanh(gates[:, 2 * HP:3 * HP])
    o = _sigmoid(gates[:, 3 * HP:4 * HP])
    c = f * c_ref[...] + i * g
    h2 = o * jnp.tanh(c)
    h_ref[...] = h2
    c_ref[...] = c

    @pl.when(t == L - 1)
    def _finish():
        logits = lax.dot_general(
            h2, Wl_ref[...], (((1,), (0,)), ((), ())),
            preferred_element_type=jnp.float32,
            precision=lax.Precision.HIGHEST,
        ) + bl_ref[...]
        m = jnp.max(logits, axis=1, keepdims=True)
        e = jnp.exp(logits - m)
        out_ref[...] = e / jnp.sum(e, axis=1, keepdims=True)


def _lstm_tc(xs, Wc, b, Wl, bl):
    return pl.pallas_call(
        _lstm_body,
        grid=(NB, L),
        in_specs=[
            pl.BlockSpec((1, BB, D), lambda i, t: (t, i, 0)),
            pl.BlockSpec((KP, GP), lambda i, t: (0, 0)),
            pl.BlockSpec((1, GP), lambda i, t: (0, 0)),
            pl.BlockSpec((HP, HP), lambda i, t: (0, 0)),
            pl.BlockSpec((1, HP), lambda i, t: (0, 0)),
        ],
        out_specs=pl.BlockSpec((BB, HP), lambda i, t: (i, 0)),
        out_shape=jax.ShapeDtypeStruct((B, HP), jnp.float32),
        scratch_shapes=[
            pltpu.VMEM((BB, HP), jnp.float32),
            pltpu.VMEM((BB, HP), jnp.float32),
        ],
        compiler_params=pltpu.CompilerParams(
            dimension_semantics=("arbitrary", "arbitrary"),
        ),
    )(xs, Wc, b, Wl, bl)


def _prep_weights(W_ih, W_hh, b_ih, b_hh, W_lin, b_lin):
    Wcat = jnp.concatenate([W_ih, W_hh], axis=1)
    Wcat = Wcat.reshape(4, H, D + H)
    Wcat = jnp.pad(Wcat, ((0, 0), (0, HP - H), (0, KP - (D + H))))
    Wc = Wcat.transpose(2, 0, 1).reshape(KP, GP)
    b4 = jnp.pad((b_ih + b_hh).reshape(4, H), ((0, 0), (0, HP - H)))
    b = b4.reshape(1, GP)
    Wl = jnp.zeros((HP, HP), jnp.float32).at[:H, :C].set(W_lin.T)
    bl = jnp.full((1, HP), -1e30, jnp.float32).at[0, :C].set(b_lin)
    return Wc, b, Wl, bl


def kernel(x, emb, W_ih, W_hh, b_ih, b_hh, W_lin, b_lin):
    idx = x.T.reshape(NW, NCH, CH)
    e_tm = _sc_gather(idx, emb)
    xs = e_tm.reshape(L, B, D)
    Wc, b, Wl, bl = _prep_weights(W_ih, W_hh, b_ih, b_hh, W_lin, b_lin)
    out = _lstm_tc(xs, Wc, b, Wl, bl)
    return out[:, :C]


# speedup vs baseline: 2.5784x; 1.3566x over previous
import functools

import jax
import jax.numpy as jnp
from jax import lax
from jax.experimental import pallas as pl
from jax.experimental.pallas import tpu as pltpu
from jax.experimental.pallas import tpu_sc as plsc

V = 100000
D = 64
H = 100
C = 6
B = 4096
L = 50

NC = 2
NS = 16
NW = NC * NS
R = B * L
ROWS_PER_W = R // NW
CH = 128
NCH = ROWS_PER_W // CH

BB = 512
NB = B // BB
HP = 128
GP = 4 * HP
KP = D + HP


def _sc_gather(idx, emb):
    mesh = plsc.VectorSubcoreMesh(core_axis_name="c", subcore_axis_name="s")

    @functools.partial(
        pl.kernel,
        mesh=mesh,
        out_type=jax.ShapeDtypeStruct((R, D), jnp.float32),
        scratch_types=[
            pltpu.VMEM((NCH, CH), jnp.int32),
            pltpu.VMEM((CH, D), jnp.float32),
            pltpu.SemaphoreType.DMA,
        ],
        compiler_params=pltpu.CompilerParams(use_tc_tiling_on_sc=False),
    )
    def k(idx_hbm, emb_hbm, out_hbm, idx_v, buf, sem):
        wid = lax.axis_index("s") * NC + lax.axis_index("c")
        base = pl.multiple_of(wid * ROWS_PER_W, CH)
        pltpu.sync_copy(idx_hbm.at[wid], idx_v)

        def body(j, carry):
            pltpu.async_copy(emb_hbm.at[idx_v.at[j]], buf, sem).wait()
            pltpu.sync_copy(buf, out_hbm.at[pl.ds(base + j * CH, CH)])
            return carry

        lax.fori_loop(0, NCH, body, 0)

    return k(idx, emb)


def _sigmoid(x):
    return 0.5 * jnp.tanh(0.5 * x) + 0.5


def _lstm_body(xs_ref, Wc_ref, b_ref, Wl_ref, bl_ref, out_ref, h_ref, c_ref):
    t = pl.program_id(1)

    @pl.when(t == 0)
    def _init():
        h_ref[...] = jnp.zeros_like(h_ref)
        c_ref[...] = jnp.zeros_like(c_ref)

    xt = xs_ref[0]
    h = h_ref[...]
    xh = jnp.concatenate(
        [xt.astype(jnp.bfloat16), h.astype(jnp.bfloat16)], axis=1)
    gates = lax.dot_general(
        xh, Wc_ref[...], (((1,), (0,)), ((), ())),
        preferred_element_type=jnp.float32,
    ) + b_ref[...]
    i = _sigmoid(gates[:, 0:HP])
    f = _sigmoid(gates[:, HP:2 * HP])
    g = jnp.tanh(gates[:, 2 * HP:3 * HP])
    o = _sigmoid(gates[:, 3 * HP:4 * HP])
    c = f * c_ref[...] + i * g
    h2 = o * jnp.tanh(c)
    h_ref[...] = h2
    c_ref[...] = c

    @pl.when(t == L - 1)
    def _finish():
        logits = lax.dot_general(
            h2.astype(jnp.bfloat16), Wl_ref[...], (((1,), (0,)), ((), ())),
            preferred_element_type=jnp.float32,
        ) + bl_ref[...]
        m = jnp.max(logits, axis=1, keepdims=True)
        e = jnp.exp(logits - m)
        out_ref[...] = e / jnp.sum(e, axis=1, keepdims=True)


def _lstm_tc(xs, Wc, b, Wl, bl):
    return pl.pallas_call(
        _lstm_body,
        grid=(NB, L),
        in_specs=[
            pl.BlockSpec((1, BB, D), lambda i, t: (t, i, 0)),
            pl.BlockSpec((KP, GP), lambda i, t: (0, 0)),
            pl.BlockSpec((1, GP), lambda i, t: (0, 0)),
            pl.BlockSpec((HP, HP), lambda i, t: (0, 0)),
            pl.BlockSpec((1, HP), lambda i, t: (0, 0)),
        ],
        out_specs=pl.BlockSpec((BB, HP), lambda i, t: (i, 0)),
        out_shape=jax.ShapeDtypeStruct((B, HP), jnp.float32),
        scratch_shapes=[
            pltpu.VMEM((BB, HP), jnp.float32),
            pltpu.VMEM((BB, HP), jnp.float32),
        ],
        compiler_params=pltpu.CompilerParams(
            dimension_semantics=("arbitrary", "arbitrary"),
        ),
    )(xs, Wc, b, Wl, bl)


def _prep_weights(W_ih, W_hh, b_ih, b_hh, W_lin, b_lin):
    Wcat = jnp.concatenate([W_ih, W_hh], axis=1)
    Wcat = Wcat.reshape(4, H, D + H)
    Wcat = jnp.pad(Wcat, ((0, 0), (0, HP - H), (0, KP - (D + H))))
    Wc = Wcat.transpose(2, 0, 1).reshape(KP, GP).astype(jnp.bfloat16)
    b4 = jnp.pad((b_ih + b_hh).reshape(4, H), ((0, 0), (0, HP - H)))
    b = b4.reshape(1, GP)
    Wl = jnp.zeros((HP, HP), jnp.bfloat16).at[:H, :C].set(W_lin.T.astype(jnp.bfloat16))
    bl = jnp.full((1, HP), -1e30, jnp.float32).at[0, :C].set(b_lin)
    return Wc, b, Wl, bl


def kernel(x, emb, W_ih, W_hh, b_ih, b_hh, W_lin, b_lin):
    idx = x.T.reshape(NW, NCH, CH)
    e_tm = _sc_gather(idx, emb)
    xs = e_tm.reshape(L, B, D)
    Wc, b, Wl, bl = _prep_weights(W_ih, W_hh, b_ih, b_hh, W_lin, b_lin)
    out = _lstm_tc(xs, Wc, b, Wl, bl)
    return out[:, :C]


# speedup vs baseline: 4.0756x; 1.5807x over previous
import functools

import jax
import jax.numpy as jnp
from jax import lax
from jax.experimental import pallas as pl
from jax.experimental.pallas import tpu as pltpu
from jax.experimental.pallas import tpu_sc as plsc

V = 100000
D = 64
H = 100
C = 6
B = 4096
L = 50

NC = 2
NS = 16
NW = NC * NS
R = B * L
ROWS_PER_W = R // NW
CH = 128
NCH = ROWS_PER_W // CH

BB = 4096
NB = B // BB
HP = 128
GP = 4 * HP
KP = D + HP


def _sc_gather(idx, emb):
    mesh = plsc.VectorSubcoreMesh(core_axis_name="c", subcore_axis_name="s")

    @functools.partial(
        pl.kernel,
        mesh=mesh,
        out_type=jax.ShapeDtypeStruct((R, D), jnp.float32),
        scratch_types=[
            pltpu.VMEM((NCH, CH), jnp.int32),
            pltpu.VMEM((CH, D), jnp.float32),
            pltpu.SemaphoreType.DMA,
        ],
        compiler_params=pltpu.CompilerParams(use_tc_tiling_on_sc=False),
    )
    def k(idx_hbm, emb_hbm, out_hbm, idx_v, buf, sem):
        wid = lax.axis_index("s") * NC + lax.axis_index("c")
        base = pl.multiple_of(wid * ROWS_PER_W, CH)
        pltpu.sync_copy(idx_hbm.at[wid], idx_v)

        def body(j, carry):
            pltpu.async_copy(emb_hbm.at[idx_v.at[j]], buf, sem).wait()
            pltpu.sync_copy(buf, out_hbm.at[pl.ds(base + j * CH, CH)])
            return carry

        lax.fori_loop(0, NCH, body, 0)

    return k(idx, emb)


def _sigmoid(x):
    return 0.5 * jnp.tanh(0.5 * x) + 0.5


def _lstm_body(xs_ref, Wc_ref, b_ref, Wl_ref, bl_ref, out_ref, h_ref, c_ref):
    t = pl.program_id(1)

    @pl.when(t == 0)
    def _init():
        h_ref[...] = jnp.zeros_like(h_ref)
        c_ref[...] = jnp.zeros_like(c_ref)

    xt = xs_ref[0]
    h = h_ref[...]
    xh = jnp.concatenate(
        [xt.astype(jnp.bfloat16), h.astype(jnp.bfloat16)], axis=1)
    gates = lax.dot_general(
        xh, Wc_ref[...], (((1,), (0,)), ((), ())),
        preferred_element_type=jnp.float32,
    ) + b_ref[...]
    i = _sigmoid(gates[:, 0:HP])
    f = _sigmoid(gates[:, HP:2 * HP])
    g = jnp.tanh(gates[:, 2 * HP:3 * HP])
    o = _sigmoid(gates[:, 3 * HP:4 * HP])
    c = f * c_ref[...] + i * g
    h2 = o * jnp.tanh(c)
    h_ref[...] = h2
    c_ref[...] = c

    @pl.when(t == L - 1)
    def _finish():
        logits = lax.dot_general(
            h2.astype(jnp.bfloat16), Wl_ref[...], (((1,), (0,)), ((), ())),
            preferred_element_type=jnp.float32,
        ) + bl_ref[...]
        m = jnp.max(logits, axis=1, keepdims=True)
        e = jnp.exp(logits - m)
        out_ref[...] = e / jnp.sum(e, axis=1, keepdims=True)


def _lstm_tc(xs, Wc, b, Wl, bl):
    return pl.pallas_call(
        _lstm_body,
        grid=(NB, L),
        in_specs=[
            pl.BlockSpec((1, BB, D), lambda i, t: (t, i, 0)),
            pl.BlockSpec((KP, GP), lambda i, t: (0, 0)),
            pl.BlockSpec((1, GP), lambda i, t: (0, 0)),
            pl.BlockSpec((HP, HP), lambda i, t: (0, 0)),
            pl.BlockSpec((1, HP), lambda i, t: (0, 0)),
        ],
        out_specs=pl.BlockSpec((BB, HP), lambda i, t: (i, 0)),
        out_shape=jax.ShapeDtypeStruct((B, HP), jnp.float32),
        scratch_shapes=[
            pltpu.VMEM((BB, HP), jnp.float32),
            pltpu.VMEM((BB, HP), jnp.float32),
        ],
        compiler_params=pltpu.CompilerParams(
            dimension_semantics=("arbitrary", "arbitrary"),
        ),
    )(xs, Wc, b, Wl, bl)


def _prep_weights(W_ih, W_hh, b_ih, b_hh, W_lin, b_lin):
    Wcat = jnp.concatenate([W_ih, W_hh], axis=1)
    Wcat = Wcat.reshape(4, H, D + H)
    Wcat = jnp.pad(Wcat, ((0, 0), (0, HP - H), (0, KP - (D + H))))
    Wc = Wcat.transpose(2, 0, 1).reshape(KP, GP).astype(jnp.bfloat16)
    b4 = jnp.pad((b_ih + b_hh).reshape(4, H), ((0, 0), (0, HP - H)))
    b = b4.reshape(1, GP)
    Wl = jnp.zeros((HP, HP), jnp.bfloat16).at[:H, :C].set(W_lin.T.astype(jnp.bfloat16))
    bl = jnp.full((1, HP), -1e30, jnp.float32).at[0, :C].set(b_lin)
    return Wc, b, Wl, bl


def kernel(x, emb, W_ih, W_hh, b_ih, b_hh, W_lin, b_lin):
    idx = x.T.reshape(NW, NCH, CH)
    e_tm = _sc_gather(idx, emb)
    xs = e_tm.reshape(L, B, D)
    Wc, b, Wl, bl = _prep_weights(W_ih, W_hh, b_ih, b_hh, W_lin, b_lin)
    out = _lstm_tc(xs, Wc, b, Wl, bl)
    return out[:, :C]
